# baseline (speedup 1.0000x reference)
import functools

import jax
import jax.numpy as jnp
from jax import lax
from jax.experimental import pallas as pl
from jax.experimental.pallas import tpu as pltpu
from jax.experimental.pallas import tpu_sc as plsc

EMBED = 64
NC = 2
NS = 16
NW = NC * NS
CHUNK = 128
NBUF = 4
LANES = 16
PADW = 128


@functools.lru_cache(maxsize=None)
def _build(rows: int, seq: int, vocab: int):
    mesh = plsc.VectorSubcoreMesh(core_axis_name="c", subcore_axis_name="s")
    rows_per_w = rows // NW
    per_w = rows_per_w * seq
    n_chunks = per_w // CHUNK
    n_groups = n_chunks // NBUF
    total = rows * seq
    assert per_w % CHUNK == 0 and n_chunks % NBUF == 0 and n_groups >= 2
    n_cvt = per_w // LANES

    pad_per_w = rows_per_w * PADW

    stage_tiles = 8
    assert vocab % stage_tiles == 0
    stage_rows = vocab // stage_tiles

    @functools.partial(
        pl.kernel,
        mesh=mesh,
        out_type=jax.ShapeDtypeStruct((total, EMBED), jnp.float32),
        scratch_types=[
            pltpu.VMEM((pad_per_w,), jnp.int32),
            pltpu.VMEM((per_w,), jnp.int32),
            pltpu.VMEM((n_chunks, CHUNK), jnp.int32),
            pltpu.VMEM((NBUF, CHUNK, EMBED), jnp.float32),
            pltpu.VMEM((stage_rows, EMBED), jnp.float32),
            pltpu.VMEM_SHARED((vocab, EMBED), jnp.float32),
            pltpu.SemaphoreType.DMA((NBUF,)),
            pltpu.SemaphoreType.DMA((NBUF,)),
        ],
        compiler_params=pltpu.CompilerParams(use_tc_tiling_on_sc=False,
                                             needs_layout_passes=False),
    )
    def emb(idx_hbm, fmap_hbm, table_hbm, out_hbm, idx_pad_v, fmap_v, idx_v,
            rows_v, tv, table_sh, gsem, osem):
        sid = lax.axis_index("s")
        wid = sid * NC + lax.axis_index("c")

        @pl.when(sid < stage_tiles)
        def _stage():
            lo = sid * stage_rows
            pltpu.sync_copy(
                table_hbm.at[pl.ds(lo, stage_rows), pl.ds(0, EMBED)], tv)
            pltpu.sync_copy(tv, table_sh.at[pl.ds(lo, stage_rows)])

        pltpu.sync_copy(idx_hbm.at[pl.ds(wid * pad_per_w, pad_per_w)],
                        idx_pad_v)
        pltpu.sync_copy(fmap_hbm, fmap_v)

        def cvt(t, carry):
            f = fmap_v[pl.ds(t * LANES, LANES)]
            v = plsc.load_gather(idx_pad_v, [f])
            idx_v[t // (CHUNK // LANES),
                  pl.ds((t % (CHUNK // LANES)) * LANES, LANES)] = v
            return carry

        lax.fori_loop(0, n_cvt, cvt, 0)
        plsc.subcore_barrier()
        base = wid * per_w

        def fire_gather(k, b):
            pltpu.async_copy(table_sh.at[idx_v.at[k]], rows_v.at[b],
                             gsem.at[b])

        def wait_gather(b):
            pltpu.make_async_copy(out_hbm.at[pl.ds(base, CHUNK)],
                                  rows_v.at[b], gsem.at[b]).wait()

        def fire_out(k, b):
            pltpu.async_copy(rows_v.at[b],
                             out_hbm.at[pl.ds(base + k * CHUNK, CHUNK)],
                             osem.at[b])

        def wait_out(b):
            pltpu.make_async_copy(rows_v.at[b],
                                  out_hbm.at[pl.ds(base, CHUNK)],
                                  osem.at[b]).wait()

        def step(k, b, prefetch):
            wait_gather(b)
            fire_out(k, b)
            if prefetch:
                bp = (b - 1) % NBUF
                wait_out(bp)
                fire_gather(k - 1 + NBUF, bp)

        for b in range(NBUF):
            fire_gather(b, b)
        for b in range(NBUF):
            step(b, b, prefetch=b > 0)

        def group(g, carry):
            for b in range(NBUF):
                step(g * NBUF + b, b, prefetch=True)
            return carry

        lax.fori_loop(1, n_groups - 1, group, 0)

        for b in range(NBUF):
            step(n_chunks - NBUF + b, b, prefetch=b == 0)
        for b in range(NBUF):
            wait_out(b)

    return emb


def kernel(inputs, table):
    batch, seq = inputs.shape
    idx_pad = jnp.pad(inputs.astype(jnp.int32),
                      ((0, 0), (0, PADW - seq))).reshape(-1)
    per_w = (batch // NW) * seq
    j = jnp.arange(per_w, dtype=jnp.int32)
    fmap = (j // seq) * PADW + j % seq
    table_pad = jnp.pad(table, ((0, 0), (0, PADW - EMBED)))
    out = _build(batch, seq, table.shape[0])(idx_pad, fmap, table_pad)
    return out.reshape(batch, seq, EMBED)
